# baseline (speedup 1.0000x reference)
import functools

import jax
import jax.numpy as jnp
from jax import lax
from jax.experimental import pallas as pl
from jax.experimental.pallas import tpu as pltpu
from jax.experimental.pallas import tpu_sc as plsc


def kernel(action_type, action_embeddings):
    (B,) = action_type.shape
    V, D = action_embeddings.shape

    info = plsc.get_sparse_core_info()
    NC, NS = info.num_cores, info.num_subcores
    NW = NC * NS
    b_per_w = B // NW
    C = 128
    n_chunks = b_per_w // C

    mesh = plsc.VectorSubcoreMesh(core_axis_name="c", subcore_axis_name="s")

    @functools.partial(
        pl.kernel,
        mesh=mesh,
        out_type=jax.ShapeDtypeStruct((B, D), jnp.float32),
        scratch_types=[
            pltpu.VMEM((2, C), jnp.int32),
            pltpu.VMEM((2, C, D), jnp.float32),
            pltpu.SemaphoreType.DMA,
            pltpu.SemaphoreType.DMA,
        ],
    )
    def gather_kernel(idx_hbm, table_hbm, out_hbm, idx_v, rows_v, gsem, osem):
        wid = lax.axis_index("s") * NC + lax.axis_index("c")
        base = wid * b_per_w
        row0 = wid * n_chunks

        def start_gather(i):
            s = i % 2
            pltpu.sync_copy(idx_hbm.at[row0 + i], idx_v.at[s])
            return pltpu.async_copy(
                table_hbm.at[idx_v.at[s]], rows_v.at[s], gsem
            )

        gcopies = {0: start_gather(0)}
        ocopies = {}
        for i in range(n_chunks):
            s = i % 2
            if i + 1 < n_chunks:
                if i >= 1:
                    ocopies[i - 1].wait()
                gcopies[i + 1] = start_gather(i + 1)
            gcopies[i].wait()
            ocopies[i] = pltpu.async_copy(
                rows_v.at[s], out_hbm.at[pl.ds(base + i * C, C)], osem
            )
        ocopies[n_chunks - 2].wait()
        ocopies[n_chunks - 1].wait()

    idx2d = action_type.astype(jnp.int32).reshape(B // C, C)
    return gather_kernel(idx2d, action_embeddings)


# speedup vs baseline: 1.5486x; 1.5486x over previous
import functools

import jax
import jax.numpy as jnp
from jax import lax
from jax.experimental import pallas as pl
from jax.experimental.pallas import tpu as pltpu
from jax.experimental.pallas import tpu_sc as plsc


def kernel(action_type, action_embeddings):
    (B,) = action_type.shape
    V, D = action_embeddings.shape

    info = plsc.get_sparse_core_info()
    NC, NS = info.num_cores, info.num_subcores
    NW = NC * NS
    b_per_w = B // NW
    C = 128
    n_chunks = b_per_w // C

    mesh = plsc.VectorSubcoreMesh(core_axis_name="c", subcore_axis_name="s")

    @functools.partial(
        pl.kernel,
        mesh=mesh,
        out_type=jax.ShapeDtypeStruct((B, D), jnp.float32),
        scratch_types=[
            pltpu.VMEM((V, D), jnp.float32),
            pltpu.VMEM((b_per_w,), jnp.int32),
            pltpu.VMEM((2, C, D), jnp.float32),
            pltpu.SemaphoreType.DMA,
        ],
    )
    def gather_kernel(idx_hbm, table_hbm, out_hbm, table_v, idx_v, obuf, osem):
        wid = lax.axis_index("s") * NC + lax.axis_index("c")
        base = wid * b_per_w
        pltpu.sync_copy(table_hbm, table_v)
        pltpu.sync_copy(idx_hbm.at[pl.ds(base, b_per_w)], idx_v)

        ocopies = {}
        for chunk in range(n_chunks):
            s = chunk % 2
            if chunk >= 2:
                ocopies[chunk - 2].wait()

            def body(g, carry, s=s, chunk=chunk):
                iv = idx_v[pl.ds(chunk * C + g * 16, 16)]
                for k in range(16):
                    row = iv[k]
                    for j0 in range(0, D // 16, 4):
                        vals = [table_v[row, pl.ds(j * 16, 16)] for j in range(j0, j0 + 4)]
                        for j in range(j0, j0 + 4):
                            obuf[s, g * 16 + k, pl.ds(j * 16, 16)] = vals[j - j0]
                return carry

            lax.fori_loop(0, C // 16, body, 0)
            ocopies[chunk] = pltpu.async_copy(
                obuf.at[s], out_hbm.at[pl.ds(base + chunk * C, C)], osem
            )
        ocopies[n_chunks - 2].wait()
        ocopies[n_chunks - 1].wait()

    return gather_kernel(action_type.astype(jnp.int32), action_embeddings)


# speedup vs baseline: 1.9969x; 1.2895x over previous
import functools

import jax
import jax.numpy as jnp
from jax import lax
from jax.experimental import pallas as pl
from jax.experimental.pallas import tpu as pltpu
from jax.experimental.pallas import tpu_sc as plsc


def kernel(action_type, action_embeddings):
    (B,) = action_type.shape
    V, D = action_embeddings.shape

    info = plsc.get_sparse_core_info()
    NC, NS = info.num_cores, info.num_subcores
    NW = NC * NS
    b_per_w = B // NW

    mesh = plsc.VectorSubcoreMesh(core_axis_name="c", subcore_axis_name="s")

    @functools.partial(
        pl.kernel,
        mesh=mesh,
        out_type=jax.ShapeDtypeStruct((B, D), jnp.float32),
        scratch_types=[
            pltpu.VMEM((V, D), jnp.float32),
            pltpu.VMEM((b_per_w,), jnp.int32),
            pltpu.SemaphoreType.DMA,
        ],
    )
    def gather_kernel(idx_hbm, table_hbm, out_hbm, table_v, idx_v, dsem):
        wid = lax.axis_index("s") * NC + lax.axis_index("c")
        base = wid * b_per_w
        pltpu.sync_copy(table_hbm, table_v)
        pltpu.sync_copy(idx_hbm.at[pl.ds(base, b_per_w)], idx_v)

        def fire(g, carry):
            iv = idx_v[pl.ds(g * 16, 16)]
            for k in range(16):
                row = iv[k]
                pltpu.async_copy(
                    table_v.at[row], out_hbm.at[base + g * 16 + k], dsem
                )
            return carry

        lax.fori_loop(0, b_per_w // 16, fire, 0)

        def drain(g, carry):
            for k in range(16):
                pltpu.make_async_copy(
                    table_v.at[0], out_hbm.at[base], dsem
                ).wait()
            return carry

        lax.fori_loop(0, b_per_w // 16, drain, 0)

    return gather_kernel(action_type.astype(jnp.int32), action_embeddings)


# speedup vs baseline: 2.2399x; 1.1217x over previous
import functools

import jax
import jax.numpy as jnp
from jax import lax
from jax.experimental import pallas as pl
from jax.experimental.pallas import tpu as pltpu
from jax.experimental.pallas import tpu_sc as plsc


def kernel(action_type, action_embeddings):
    (B,) = action_type.shape
    V, D = action_embeddings.shape

    info = plsc.get_sparse_core_info()
    NC, NS = info.num_cores, info.num_subcores
    NW = NC * NS
    b_per_w = B // NW

    mesh = plsc.VectorSubcoreMesh(core_axis_name="c", subcore_axis_name="s")

    @functools.partial(
        pl.kernel,
        mesh=mesh,
        out_type=jax.ShapeDtypeStruct((B, D), jnp.float32),
        scratch_types=[
            pltpu.VMEM_SHARED((V, D), jnp.float32),
            pltpu.VMEM((V, D), jnp.float32),
            pltpu.VMEM((b_per_w,), jnp.int32),
            pltpu.SemaphoreType.DMA,
        ],
    )
    def gather_kernel(idx_hbm, table_hbm, out_hbm, table_sh, table_v, idx_v, dsem):
        sid = lax.axis_index("s")
        wid = sid * NC + lax.axis_index("c")
        base = wid * b_per_w

        @pl.when(sid == 0)
        def _():
            pltpu.sync_copy(table_hbm, table_sh)

        pltpu.sync_copy(idx_hbm.at[pl.ds(base, b_per_w)], idx_v)
        plsc.subcore_barrier()
        pltpu.sync_copy(table_sh, table_v)

        def fire(g, carry):
            iv = idx_v[pl.ds(g * 16, 16)]
            for k in range(16):
                row = iv[k]
                pltpu.async_copy(
                    table_v.at[row], out_hbm.at[base + g * 16 + k], dsem
                )
            return carry

        lax.fori_loop(0, b_per_w // 16, fire, 0)

        def drain(g, carry):
            for k in range(16):
                pltpu.make_async_copy(
                    table_v.at[0], out_hbm.at[base], dsem
                ).wait()
            return carry

        lax.fori_loop(0, b_per_w // 16, drain, 0)

    return gather_kernel(action_type.astype(jnp.int32), action_embeddings)
